# baseline (speedup 1.0000x reference)
import functools

import jax
import jax.numpy as jnp
from jax import lax
from jax.experimental import pallas as pl
from jax.experimental.pallas import tpu as pltpu
from jax.experimental.pallas import tpu_sc as plsc

N_NODES = 10000
D_FEAT = 128
N_EDGES = 320000

NC = 2
NS = 16
NW = NC * NS

N_PAD = 10240
ROWS_PER_TILE = N_PAD // NS
DUMP_ROW = N_NODES + 128

CHUNK = 80
NBUF = 2
CPT = 128
E_PAD = NW * CPT * CHUNK


@functools.partial(
    pl.kernel,
    out_type=(
        jax.ShapeDtypeStruct((NC * N_PAD, D_FEAT), jnp.float32),
        jax.ShapeDtypeStruct((NC * N_PAD,), jnp.float32),
    ),
    mesh=plsc.VectorSubcoreMesh(
        core_axis_name="c", subcore_axis_name="s", num_cores=NC, num_subcores=NS
    ),
    scratch_types=(
        [pltpu.VMEM((CHUNK,), jnp.int32) for _ in range(NBUF)]
        + [pltpu.VMEM((CHUNK,), jnp.int32) for _ in range(NBUF)]
        + [pltpu.VMEM((CHUNK, D_FEAT), jnp.float32) for _ in range(NBUF)]
        + [
            pltpu.VMEM((CHUNK,), jnp.float32),
            pltpu.VMEM((ROWS_PER_TILE,), jnp.float32),
            pltpu.VMEM_SHARED((N_PAD, D_FEAT), jnp.float32),
            pltpu.VMEM_SHARED((N_PAD,), jnp.float32),
        ]
        + [pltpu.SemaphoreType.DMA for _ in range(NBUF)]
    ),
)
def _sc_aggregate(x_hbm, src_hbm, dst_hbm, zrows_hbm, ones_hbm, zdeg_hbm,
                  part_out, deg_out, *refs):
    srcs = refs[0:NBUF]
    dsts = refs[NBUF:2 * NBUF]
    bufs = refs[2 * NBUF:3 * NBUF]
    ones_v, dstage_v, acc_sh, dacc_sh = refs[3 * NBUF:3 * NBUF + 4]
    sems = refs[3 * NBUF + 4:]

    c = lax.axis_index("c")
    s = lax.axis_index("s")
    w = c * NS + s

    pltpu.sync_copy(zrows_hbm.at[pl.ds(0, CHUNK), :], bufs[0])
    pltpu.sync_copy(ones_hbm.at[pl.ds(0, CHUNK)], ones_v)
    pltpu.sync_copy(zdeg_hbm.at[pl.ds(0, ROWS_PER_TILE)], dstage_v)

    row0 = s * ROWS_PER_TILE
    for k in range(ROWS_PER_TILE // CHUNK):
        pltpu.sync_copy(bufs[0], acc_sh.at[pl.ds(row0 + k * CHUNK, CHUNK), :])
    pltpu.sync_copy(dstage_v, dacc_sh.at[pl.ds(row0, ROWS_PER_TILE)])
    plsc.subcore_barrier()

    ebase = w * CPT * CHUNK

    for b in range(NBUF):
        e = ebase + b * CHUNK
        pltpu.sync_copy(src_hbm.at[pl.ds(e, CHUNK)], srcs[b])
        pltpu.sync_copy(dst_hbm.at[pl.ds(e, CHUNK)], dsts[b])
        pltpu.async_copy(x_hbm.at[srcs[b]], bufs[b], sems[b])

    def ring_body(i, carry):
        for b in range(NBUF):
            g = NBUF * i + b
            pltpu.make_async_copy(x_hbm.at[srcs[b]], bufs[b], sems[b]).wait()
            pltpu.sync_copy(bufs[b], acc_sh.at[dsts[b]], add=True)
            pltpu.sync_copy(ones_v, dacc_sh.at[dsts[b]], add=True)
            e = ebase + (g + NBUF) * CHUNK
            pltpu.sync_copy(src_hbm.at[pl.ds(e, CHUNK)], srcs[b])
            pltpu.sync_copy(dst_hbm.at[pl.ds(e, CHUNK)], dsts[b])
            pltpu.async_copy(x_hbm.at[srcs[b]], bufs[b], sems[b])
        return carry

    lax.fori_loop(0, (CPT - NBUF) // NBUF, ring_body, 0)

    for b in range(NBUF):
        pltpu.make_async_copy(x_hbm.at[srcs[b]], bufs[b], sems[b]).wait()
        pltpu.sync_copy(bufs[b], acc_sh.at[dsts[b]], add=True)
        pltpu.sync_copy(ones_v, dacc_sh.at[dsts[b]], add=True)

    plsc.subcore_barrier()

    obase = c * N_PAD + row0
    for k in range(ROWS_PER_TILE // CHUNK):
        r = row0 + k * CHUNK
        o = obase + k * CHUNK
        pltpu.sync_copy(acc_sh.at[pl.ds(r, CHUNK), :], bufs[0])
        pltpu.sync_copy(bufs[0], part_out.at[pl.ds(o, CHUNK), :])
    pltpu.sync_copy(dacc_sh.at[pl.ds(row0, ROWS_PER_TILE)], dstage_v)
    pltpu.sync_copy(dstage_v, deg_out.at[pl.ds(obase, ROWS_PER_TILE)])


_BLK = 512


def _combine_body(p_ref, d_ref, o_ref):
    p0 = p_ref[0]
    p1 = p_ref[1]
    deg = d_ref[0] + d_ref[1]
    o_ref[...] = (p0 + p1) / jnp.maximum(deg, 1.0)


_combine = pl.pallas_call(
    _combine_body,
    grid=(20,),
    in_specs=[
        pl.BlockSpec((NC, _BLK, D_FEAT), lambda i: (0, i, 0)),
        pl.BlockSpec((NC, _BLK, 1), lambda i: (0, i, 0)),
    ],
    out_specs=pl.BlockSpec((_BLK, D_FEAT), lambda i: (i, 0)),
    out_shape=jax.ShapeDtypeStruct((N_NODES, D_FEAT), jnp.float32),
)


def kernel(x, edge_index):
    src = edge_index[0].astype(jnp.int32)
    dst = edge_index[1].astype(jnp.int32)
    npad = E_PAD - N_EDGES
    src = jnp.concatenate([src, jnp.zeros((npad,), jnp.int32)])
    dst = jnp.concatenate([dst, jnp.full((npad,), DUMP_ROW, jnp.int32)])
    zrows = jnp.zeros((CHUNK, D_FEAT), jnp.float32)
    ones = jnp.ones((CHUNK,), jnp.float32)
    zdeg = jnp.zeros((N_PAD,), jnp.float32)
    part, deg = _sc_aggregate(x, src, dst, zrows, ones, zdeg)
    part = part.reshape(NC, N_PAD, D_FEAT)
    deg = deg.reshape(NC, N_PAD, 1)
    return _combine(part, deg)


# speedup vs baseline: 2.2147x; 2.2147x over previous
import functools

import jax
import jax.numpy as jnp
from jax import lax
from jax.experimental import pallas as pl
from jax.experimental.pallas import tpu as pltpu
from jax.experimental.pallas import tpu_sc as plsc

N_NODES = 10000
D_FEAT = 128
N_EDGES = 320000

NC = 2
NS = 16
NW = NC * NS

N_PAD = 10240
ROWS_PER_TILE = N_PAD // NS
DUMP_ROW = N_NODES + 128

CHUNK = 80
NBUF = 4
CPT = 128
E_PAD = NW * CPT * CHUNK


@functools.partial(
    pl.kernel,
    out_type=(
        jax.ShapeDtypeStruct((NC * N_PAD, D_FEAT), jnp.float32),
        jax.ShapeDtypeStruct((NC * N_PAD,), jnp.float32),
    ),
    mesh=plsc.VectorSubcoreMesh(
        core_axis_name="c", subcore_axis_name="s", num_cores=NC, num_subcores=NS
    ),
    scratch_types=(
        [pltpu.VMEM((CHUNK,), jnp.int32) for _ in range(NBUF)]
        + [pltpu.VMEM((CHUNK,), jnp.int32) for _ in range(NBUF)]
        + [pltpu.VMEM((CHUNK, D_FEAT), jnp.float32) for _ in range(NBUF)]
        + [
            pltpu.VMEM((CHUNK,), jnp.float32),
            pltpu.VMEM((ROWS_PER_TILE,), jnp.float32),
            pltpu.VMEM_SHARED((N_PAD, D_FEAT), jnp.float32),
            pltpu.VMEM_SHARED((N_PAD,), jnp.float32),
        ]
        + [pltpu.SemaphoreType.DMA for _ in range(NBUF)]
    ),
)
def _sc_aggregate(x_hbm, src_hbm, dst_hbm, zrows_hbm, ones_hbm, zdeg_hbm,
                  part_out, deg_out, *refs):
    srcs = refs[0:NBUF]
    dsts = refs[NBUF:2 * NBUF]
    bufs = refs[2 * NBUF:3 * NBUF]
    ones_v, dstage_v, acc_sh, dacc_sh = refs[3 * NBUF:3 * NBUF + 4]
    sems = refs[3 * NBUF + 4:]

    c = lax.axis_index("c")
    s = lax.axis_index("s")
    w = c * NS + s

    pltpu.sync_copy(zrows_hbm.at[pl.ds(0, CHUNK), :], bufs[0])
    pltpu.sync_copy(ones_hbm.at[pl.ds(0, CHUNK)], ones_v)
    pltpu.sync_copy(zdeg_hbm.at[pl.ds(0, ROWS_PER_TILE)], dstage_v)

    row0 = s * ROWS_PER_TILE
    for k in range(ROWS_PER_TILE // CHUNK):
        pltpu.sync_copy(bufs[0], acc_sh.at[pl.ds(row0 + k * CHUNK, CHUNK), :])
    pltpu.sync_copy(dstage_v, dacc_sh.at[pl.ds(row0, ROWS_PER_TILE)])
    plsc.subcore_barrier()

    ebase = w * CPT * CHUNK

    for b in range(NBUF):
        e = ebase + b * CHUNK
        pltpu.sync_copy(src_hbm.at[pl.ds(e, CHUNK)], srcs[b])
        pltpu.sync_copy(dst_hbm.at[pl.ds(e, CHUNK)], dsts[b])
        pltpu.async_copy(x_hbm.at[srcs[b]], bufs[b], sems[b])

    def ring_body(i, carry):
        for b in range(NBUF):
            g = NBUF * i + b
            pltpu.make_async_copy(x_hbm.at[srcs[b]], bufs[b], sems[b]).wait()
            pltpu.sync_copy(bufs[b], acc_sh.at[dsts[b]], add=True)
            pltpu.sync_copy(ones_v, dacc_sh.at[dsts[b]], add=True)
            e = ebase + (g + NBUF) * CHUNK
            pltpu.sync_copy(src_hbm.at[pl.ds(e, CHUNK)], srcs[b])
            pltpu.sync_copy(dst_hbm.at[pl.ds(e, CHUNK)], dsts[b])
            pltpu.async_copy(x_hbm.at[srcs[b]], bufs[b], sems[b])
        return carry

    lax.fori_loop(0, (CPT - NBUF) // NBUF, ring_body, 0)

    for b in range(NBUF):
        pltpu.make_async_copy(x_hbm.at[srcs[b]], bufs[b], sems[b]).wait()
        pltpu.sync_copy(bufs[b], acc_sh.at[dsts[b]], add=True)
        pltpu.sync_copy(ones_v, dacc_sh.at[dsts[b]], add=True)

    plsc.subcore_barrier()

    obase = c * N_PAD + row0
    for k in range(ROWS_PER_TILE // CHUNK):
        r = row0 + k * CHUNK
        o = obase + k * CHUNK
        pltpu.sync_copy(acc_sh.at[pl.ds(r, CHUNK), :], bufs[0])
        pltpu.sync_copy(bufs[0], part_out.at[pl.ds(o, CHUNK), :])
    pltpu.sync_copy(dacc_sh.at[pl.ds(row0, ROWS_PER_TILE)], dstage_v)
    pltpu.sync_copy(dstage_v, deg_out.at[pl.ds(obase, ROWS_PER_TILE)])


_BLK = 512


def _combine_body(p_ref, d_ref, o_ref):
    p0 = p_ref[0]
    p1 = p_ref[1]
    deg = d_ref[0] + d_ref[1]
    o_ref[...] = (p0 + p1) / jnp.maximum(deg, 1.0)


_combine = pl.pallas_call(
    _combine_body,
    grid=(20,),
    in_specs=[
        pl.BlockSpec((NC, _BLK, D_FEAT), lambda i: (0, i, 0)),
        pl.BlockSpec((NC, _BLK, 1), lambda i: (0, i, 0)),
    ],
    out_specs=pl.BlockSpec((_BLK, D_FEAT), lambda i: (i, 0)),
    out_shape=jax.ShapeDtypeStruct((N_NODES, D_FEAT), jnp.float32),
)


def kernel(x, edge_index):
    src = edge_index[0].astype(jnp.int32)
    dst = edge_index[1].astype(jnp.int32)
    npad = E_PAD - N_EDGES
    pidx = jnp.arange(npad, dtype=jnp.int32)
    src = jnp.concatenate([src, pidx % N_NODES])
    dst = jnp.concatenate([dst, DUMP_ROW + (pidx % (N_PAD - DUMP_ROW))])
    zrows = jnp.zeros((CHUNK, D_FEAT), jnp.float32)
    ones = jnp.ones((CHUNK,), jnp.float32)
    zdeg = jnp.zeros((N_PAD,), jnp.float32)
    part, deg = _sc_aggregate(x, src, dst, zrows, ones, zdeg)
    part = part.reshape(NC, N_PAD, D_FEAT)
    deg = deg.reshape(NC, N_PAD, 1)
    return _combine(part, deg)


# speedup vs baseline: 2.6094x; 1.1783x over previous
import functools

import jax
import jax.numpy as jnp
from jax import lax
from jax.experimental import pallas as pl
from jax.experimental.pallas import tpu as pltpu
from jax.experimental.pallas import tpu_sc as plsc

N_NODES = 10000
D_FEAT = 128
N_EDGES = 320000

NC = 2
NS = 16
NW = NC * NS

N_PAD = 10240
ROWS_PER_TILE = N_PAD // NS
DUMP_ROW = N_NODES + 128

CHUNK = 128
NBUF = 2
CPT = 80
E_PAD = NW * CPT * CHUNK


@functools.partial(
    pl.kernel,
    out_type=(
        jax.ShapeDtypeStruct((NC * N_PAD, D_FEAT), jnp.float32),
        jax.ShapeDtypeStruct((NC * N_PAD,), jnp.float32),
    ),
    mesh=plsc.VectorSubcoreMesh(
        core_axis_name="c", subcore_axis_name="s", num_cores=NC, num_subcores=NS
    ),
    scratch_types=(
        [pltpu.VMEM((CHUNK,), jnp.int32) for _ in range(NBUF)]
        + [pltpu.VMEM((CHUNK,), jnp.int32) for _ in range(NBUF)]
        + [pltpu.VMEM((CHUNK, D_FEAT), jnp.float32) for _ in range(NBUF)]
        + [
            pltpu.VMEM((CHUNK,), jnp.float32),
            pltpu.VMEM((ROWS_PER_TILE,), jnp.float32),
            pltpu.VMEM_SHARED((N_PAD, D_FEAT), jnp.float32),
            pltpu.VMEM_SHARED((N_PAD,), jnp.float32),
        ]
        + [pltpu.SemaphoreType.DMA for _ in range(NBUF)]
    ),
)
def _sc_aggregate(x_hbm, src_hbm, dst_hbm, zrows_hbm, ones_hbm, zdeg_hbm,
                  part_out, deg_out, *refs):
    srcs = refs[0:NBUF]
    dsts = refs[NBUF:2 * NBUF]
    bufs = refs[2 * NBUF:3 * NBUF]
    ones_v, dstage_v, acc_sh, dacc_sh = refs[3 * NBUF:3 * NBUF + 4]
    sems = refs[3 * NBUF + 4:]

    c = lax.axis_index("c")
    s = lax.axis_index("s")
    w = c * NS + s

    pltpu.sync_copy(zrows_hbm.at[pl.ds(0, CHUNK), :], bufs[0])
    pltpu.sync_copy(ones_hbm.at[pl.ds(0, CHUNK)], ones_v)
    pltpu.sync_copy(zdeg_hbm.at[pl.ds(0, ROWS_PER_TILE)], dstage_v)

    row0 = s * ROWS_PER_TILE
    for k in range(ROWS_PER_TILE // CHUNK):
        pltpu.sync_copy(bufs[0], acc_sh.at[pl.ds(row0 + k * CHUNK, CHUNK), :])
    pltpu.sync_copy(dstage_v, dacc_sh.at[pl.ds(row0, ROWS_PER_TILE)])
    plsc.subcore_barrier()

    ebase = w * CPT * CHUNK

    for b in range(NBUF):
        e = ebase + b * CHUNK
        pltpu.sync_copy(src_hbm.at[pl.ds(e, CHUNK)], srcs[b])
        pltpu.sync_copy(dst_hbm.at[pl.ds(e, CHUNK)], dsts[b])
        pltpu.async_copy(x_hbm.at[srcs[b]], bufs[b], sems[b])

    def ring_body(i, carry):
        for b in range(NBUF):
            g = NBUF * i + b
            pltpu.make_async_copy(x_hbm.at[srcs[b]], bufs[b], sems[b]).wait()
            pltpu.sync_copy(bufs[b], acc_sh.at[dsts[b]], add=True)
            pltpu.sync_copy(ones_v, dacc_sh.at[dsts[b]], add=True)
            e = ebase + (g + NBUF) * CHUNK
            pltpu.sync_copy(src_hbm.at[pl.ds(e, CHUNK)], srcs[b])
            pltpu.sync_copy(dst_hbm.at[pl.ds(e, CHUNK)], dsts[b])
            pltpu.async_copy(x_hbm.at[srcs[b]], bufs[b], sems[b])
        return carry

    lax.fori_loop(0, (CPT - NBUF) // NBUF, ring_body, 0)

    for b in range(NBUF):
        pltpu.make_async_copy(x_hbm.at[srcs[b]], bufs[b], sems[b]).wait()
        pltpu.sync_copy(bufs[b], acc_sh.at[dsts[b]], add=True)
        pltpu.sync_copy(ones_v, dacc_sh.at[dsts[b]], add=True)

    plsc.subcore_barrier()

    obase = c * N_PAD + row0
    for k in range(ROWS_PER_TILE // CHUNK):
        r = row0 + k * CHUNK
        o = obase + k * CHUNK
        pltpu.sync_copy(acc_sh.at[pl.ds(r, CHUNK), :], bufs[0])
        pltpu.sync_copy(bufs[0], part_out.at[pl.ds(o, CHUNK), :])
    pltpu.sync_copy(dacc_sh.at[pl.ds(row0, ROWS_PER_TILE)], dstage_v)
    pltpu.sync_copy(dstage_v, deg_out.at[pl.ds(obase, ROWS_PER_TILE)])


_BLK = 512


def _combine_body(p_ref, d_ref, o_ref):
    p0 = p_ref[0]
    p1 = p_ref[1]
    deg = d_ref[0] + d_ref[1]
    o_ref[...] = (p0 + p1) / jnp.maximum(deg, 1.0)


_combine = pl.pallas_call(
    _combine_body,
    grid=(20,),
    in_specs=[
        pl.BlockSpec((NC, _BLK, D_FEAT), lambda i: (0, i, 0)),
        pl.BlockSpec((NC, _BLK, 1), lambda i: (0, i, 0)),
    ],
    out_specs=pl.BlockSpec((_BLK, D_FEAT), lambda i: (i, 0)),
    out_shape=jax.ShapeDtypeStruct((N_NODES, D_FEAT), jnp.float32),
)


def kernel(x, edge_index):
    src = edge_index[0].astype(jnp.int32)
    dst = edge_index[1].astype(jnp.int32)
    npad = E_PAD - N_EDGES
    pidx = jnp.arange(npad, dtype=jnp.int32)
    src = jnp.concatenate([src, pidx % N_NODES])
    dst = jnp.concatenate([dst, DUMP_ROW + (pidx % (N_PAD - DUMP_ROW))])
    zrows = jnp.zeros((CHUNK, D_FEAT), jnp.float32)
    ones = jnp.ones((CHUNK,), jnp.float32)
    zdeg = jnp.zeros((N_PAD,), jnp.float32)
    part, deg = _sc_aggregate(x, src, dst, zrows, ones, zdeg)
    part = part.reshape(NC, N_PAD, D_FEAT)
    deg = deg.reshape(NC, N_PAD, 1)
    return _combine(part, deg)
